# baseline (speedup 1.0000x reference)
import functools

import jax
import jax.numpy as jnp
from jax import lax
from jax.experimental import pallas as pl
from jax.experimental.pallas import tpu as pltpu
from jax.experimental.pallas import tpu_sc as plsc

B, S, D, NS, MW = 16, 2048, 1024, 512, 12
LANES = 16
TS = 4096
CPAD = 128


def _sc_span_counts(starts, ends, smask, slab):
    mesh = plsc.VectorSubcoreMesh(core_axis_name="c", subcore_axis_name="s")
    info = plsc.get_sparse_core_info()
    nc = info.num_cores

    @functools.partial(
        pl.kernel,
        mesh=mesh,
        out_type=jax.ShapeDtypeStruct((B, S), jnp.int32),
        compiler_params=pltpu.CompilerParams(needs_layout_passes=False),
        scratch_types=[
            pltpu.VMEM((NS,), jnp.int32),
            pltpu.VMEM((NS,), jnp.int32),
            pltpu.VMEM((NS,), jnp.int32),
            pltpu.VMEM((NS,), jnp.int32),
            pltpu.VMEM((S,), jnp.int32),
        ],
    )
    def sc_kernel(st_hbm, en_hbm, mk_hbm, lb_hbm, out_hbm, st_v, en_v, mk_v, lb_v, cnt_v):
        wid = lax.axis_index("s") * nc + lax.axis_index("c")

        @pl.when(wid < B)
        def _():
            bidx = wid
            pltpu.sync_copy(st_hbm.at[bidx], st_v)
            pltpu.sync_copy(en_hbm.at[bidx], en_v)
            pltpu.sync_copy(mk_hbm.at[bidx], mk_v)
            pltpu.sync_copy(lb_hbm.at[bidx], lb_v)

            def zero_body(i, carry):
                cnt_v[pl.ds(i * LANES, LANES)] = jnp.zeros((LANES,), jnp.int32)
                return carry

            lax.fori_loop(0, S // LANES, zero_body, 0)

            def span_body(i, carry):
                sl = pl.ds(i * LANES, LANES)
                st = st_v[sl]
                last = en_v[sl] - 1
                valid = (mk_v[sl] != 0) & (lb_v[sl] > 0)
                single = last == st
                val_start = jnp.where(
                    valid,
                    jnp.where(single, jnp.int32(1 << 20), jnp.int32(1)),
                    jnp.int32(0),
                )
                val_last = jnp.where(
                    valid & (~single), jnp.int32(1 << 10), jnp.int32(0)
                )
                plsc.addupdate_scatter(cnt_v, [st], val_start)
                plsc.addupdate_scatter(cnt_v, [last], val_last)
                return carry

            lax.fori_loop(0, NS // LANES, span_body, 0)
            pltpu.sync_copy(cnt_v, out_hbm.at[bidx])

    return sc_kernel(starts, ends, smask, slab)


def _tc_loss_body(x_ref, wp_ref, bp_ref, cnt_ref, msk_ref, out_ref, acc_ref):
    i = pl.program_id(0)

    @pl.when(i == 0)
    def _():
        acc_ref[0] = jnp.float32(0.0)
        acc_ref[1] = jnp.float32(0.0)

    x = x_ref[...]
    logits = (
        jnp.dot(x, wp_ref[...], preferred_element_type=jnp.float32)
        + bp_ref[...]
    )

    ci = lax.broadcasted_iota(jnp.int32, (TS, CPAD), 1)
    is_cls = ci < 4
    neg = jnp.float32(-1e30)
    mx = jnp.max(jnp.where(is_cls, logits, neg), axis=1, keepdims=True)
    ex = jnp.where(is_cls, jnp.exp(logits - mx), 0.0)
    lse = mx + jnp.log(jnp.sum(ex, axis=1, keepdims=True))

    v = cnt_ref[...]
    c_single = (v >> 20) & 1023
    c_last = (v >> 10) & 1023
    c_start = v & 1023
    lab = jnp.where(
        c_single > 0,
        jnp.int32(2),
        jnp.where(c_last > 0, jnp.int32(1), jnp.where(c_start > 0, jnp.int32(0), jnp.int32(3))),
    )
    sel = jnp.sum(jnp.where(ci == lab, logits, 0.0), axis=1, keepdims=True)

    m = msk_ref[...]
    acc_ref[0] += jnp.sum((lse - sel) * m)
    acc_ref[1] += jnp.sum(m)

    @pl.when(i == pl.num_programs(0) - 1)
    def _():
        out_ref[0, 0] = acc_ref[0] / jnp.maximum(acc_ref[1], 1.0)


def kernel(token_reps, token_masks, span_ids, span_masks, span_labels, W, b):
    starts = span_ids[..., 0].astype(jnp.int32)
    ends = span_ids[..., 1].astype(jnp.int32)
    smask = span_masks.astype(jnp.int32)
    slab = span_labels.astype(jnp.int32)

    counts = _sc_span_counts(starts, ends, smask, slab)

    x = token_reps.reshape(B * S, D)
    wp = jnp.pad(W.T.astype(jnp.float32), ((0, 0), (0, CPAD - 4)))
    bp = jnp.pad(b.astype(jnp.float32).reshape(1, 4), ((0, 0), (0, CPAD - 4)))
    cnt2 = counts.reshape(B * S, 1)
    msk2 = token_masks.astype(jnp.float32).reshape(B * S, 1)

    nb = (B * S) // TS
    out = pl.pallas_call(
        _tc_loss_body,
        grid=(nb,),
        in_specs=[
            pl.BlockSpec((TS, D), lambda i: (i, 0)),
            pl.BlockSpec((D, CPAD), lambda i: (0, 0)),
            pl.BlockSpec((1, CPAD), lambda i: (0, 0)),
            pl.BlockSpec((TS, 1), lambda i: (i, 0)),
            pl.BlockSpec((TS, 1), lambda i: (i, 0)),
        ],
        out_specs=pl.BlockSpec(memory_space=pltpu.MemorySpace.SMEM),
        out_shape=jax.ShapeDtypeStruct((1, 1), jnp.float32),
        scratch_shapes=[pltpu.SMEM((2,), jnp.float32)],
    )(x, wp, bp, cnt2, msk2)

    return out[0, 0]


# speedup vs baseline: 1.0111x; 1.0111x over previous
import functools

import jax
import jax.numpy as jnp
from jax import lax
from jax.experimental import pallas as pl
from jax.experimental.pallas import tpu as pltpu
from jax.experimental.pallas import tpu_sc as plsc

B, S, D, NS, MW = 16, 2048, 1024, 512, 12
LANES = 16
TS = 2048
CPAD = 128


def _sc_span_counts(starts, ends, smask, slab):
    mesh = plsc.VectorSubcoreMesh(core_axis_name="c", subcore_axis_name="s")
    info = plsc.get_sparse_core_info()
    nc = info.num_cores

    @functools.partial(
        pl.kernel,
        mesh=mesh,
        out_type=jax.ShapeDtypeStruct((B, S), jnp.int32),
        compiler_params=pltpu.CompilerParams(needs_layout_passes=False),
        scratch_types=[
            pltpu.VMEM((NS,), jnp.int32),
            pltpu.VMEM((NS,), jnp.int32),
            pltpu.VMEM((NS,), jnp.int32),
            pltpu.VMEM((NS,), jnp.int32),
            pltpu.VMEM((S,), jnp.int32),
        ],
    )
    def sc_kernel(st_hbm, en_hbm, mk_hbm, lb_hbm, out_hbm, st_v, en_v, mk_v, lb_v, cnt_v):
        wid = lax.axis_index("s") * nc + lax.axis_index("c")

        @pl.when(wid < B)
        def _():
            bidx = wid
            pltpu.sync_copy(st_hbm.at[bidx], st_v)
            pltpu.sync_copy(en_hbm.at[bidx], en_v)
            pltpu.sync_copy(mk_hbm.at[bidx], mk_v)
            pltpu.sync_copy(lb_hbm.at[bidx], lb_v)

            def zero_body(i, carry):
                cnt_v[pl.ds(i * LANES, LANES)] = jnp.zeros((LANES,), jnp.int32)
                return carry

            lax.fori_loop(0, S // LANES, zero_body, 0)

            def span_body(i, carry):
                sl = pl.ds(i * LANES, LANES)
                st = st_v[sl]
                last = en_v[sl] - 1
                valid = (mk_v[sl] != 0) & (lb_v[sl] > 0)
                single = last == st
                val_start = jnp.where(
                    valid,
                    jnp.where(single, jnp.int32(1 << 20), jnp.int32(1)),
                    jnp.int32(0),
                )
                val_last = jnp.where(
                    valid & (~single), jnp.int32(1 << 10), jnp.int32(0)
                )
                plsc.addupdate_scatter(cnt_v, [st], val_start)
                plsc.addupdate_scatter(cnt_v, [last], val_last)
                return carry

            lax.fori_loop(0, NS // LANES, span_body, 0)
            pltpu.sync_copy(cnt_v, out_hbm.at[bidx])

    return sc_kernel(starts, ends, smask, slab)


def _tc_loss_body(x_ref, wp_ref, bp_ref, cnt_ref, msk_ref, out_ref, acc_ref):
    i = pl.program_id(0)

    @pl.when(i == 0)
    def _():
        acc_ref[0] = jnp.float32(0.0)
        acc_ref[1] = jnp.float32(0.0)

    x = x_ref[...]
    logits = (
        jnp.dot(x, wp_ref[...], preferred_element_type=jnp.float32)
        + bp_ref[...]
    )

    ci = lax.broadcasted_iota(jnp.int32, (TS, CPAD), 1)
    is_cls = ci < 4
    neg = jnp.float32(-1e30)
    mx = jnp.max(jnp.where(is_cls, logits, neg), axis=1, keepdims=True)
    ex = jnp.where(is_cls, jnp.exp(logits - mx), 0.0)
    lse = mx + jnp.log(jnp.sum(ex, axis=1, keepdims=True))

    v = cnt_ref[...]
    c_single = (v >> 20) & 1023
    c_last = (v >> 10) & 1023
    c_start = v & 1023
    lab = jnp.where(
        c_single > 0,
        jnp.int32(2),
        jnp.where(c_last > 0, jnp.int32(1), jnp.where(c_start > 0, jnp.int32(0), jnp.int32(3))),
    )
    sel = jnp.sum(jnp.where(ci == lab, logits, 0.0), axis=1, keepdims=True)

    m = msk_ref[...]
    acc_ref[0] += jnp.sum((lse - sel) * m)
    acc_ref[1] += jnp.sum(m)

    @pl.when(i == pl.num_programs(0) - 1)
    def _():
        out_ref[0, 0] = acc_ref[0] / jnp.maximum(acc_ref[1], 1.0)


def kernel(token_reps, token_masks, span_ids, span_masks, span_labels, W, b):
    starts = span_ids[..., 0].astype(jnp.int32)
    ends = span_ids[..., 1].astype(jnp.int32)
    smask = span_masks.astype(jnp.int32)
    slab = span_labels.astype(jnp.int32)

    counts = _sc_span_counts(starts, ends, smask, slab)

    x = token_reps.reshape(B * S, D)
    wp = jnp.pad(W.T.astype(jnp.float32), ((0, 0), (0, CPAD - 4)))
    bp = jnp.pad(b.astype(jnp.float32).reshape(1, 4), ((0, 0), (0, CPAD - 4)))
    cnt2 = counts.reshape(B * S, 1)
    msk2 = token_masks.astype(jnp.float32).reshape(B * S, 1)

    nb = (B * S) // TS
    out = pl.pallas_call(
        _tc_loss_body,
        grid=(nb,),
        in_specs=[
            pl.BlockSpec((TS, D), lambda i: (i, 0)),
            pl.BlockSpec((D, CPAD), lambda i: (0, 0)),
            pl.BlockSpec((1, CPAD), lambda i: (0, 0)),
            pl.BlockSpec((TS, 1), lambda i: (i, 0)),
            pl.BlockSpec((TS, 1), lambda i: (i, 0)),
        ],
        out_specs=pl.BlockSpec(memory_space=pltpu.MemorySpace.SMEM),
        out_shape=jax.ShapeDtypeStruct((1, 1), jnp.float32),
        scratch_shapes=[pltpu.SMEM((2,), jnp.float32)],
    )(x, wp, bp, cnt2, msk2)

    return out[0, 0]
